# baseline (speedup 1.0000x reference)
import jax
import jax.numpy as jnp
from jax.experimental import pallas as pl
from jax.experimental.pallas import tpu as pltpu

N = 10000
BR = 200


def _pass1_kernel(x_ref, a_ref, w0_ref, w1_ref, g_ref, h0_ref):
    @pl.when(pl.program_id(0) == 0)
    def _():
        h0_ref[...] = jnp.dot(
            x_ref[...], w0_ref[...],
            preferred_element_type=jnp.float32,
            precision=jax.lax.Precision.HIGHEST,
        )

    z = jnp.dot(
        a_ref[...], h0_ref[...],
        preferred_element_type=jnp.float32,
        precision=jax.lax.Precision.HIGHEST,
    )
    z = jnp.maximum(z, 0.0)
    g_ref[...] = jnp.dot(
        z, w1_ref[...],
        preferred_element_type=jnp.float32,
        precision=jax.lax.Precision.HIGHEST,
    )


def _pass2_kernel(a_ref, g_ref, out_ref):
    logits = jnp.dot(
        a_ref[...], g_ref[...],
        preferred_element_type=jnp.float32,
        precision=jax.lax.Precision.HIGHEST,
    )
    m = jnp.max(logits, axis=-1, keepdims=True)
    e = jnp.exp(logits - m)
    out_ref[...] = e / jnp.sum(e, axis=-1, keepdims=True)


def kernel(x, a, W0, W1):
    n, f_in = x.shape
    c0 = W0.shape[1]
    c1 = W1.shape[1]
    grid = (n // BR,)

    g = pl.pallas_call(
        _pass1_kernel,
        grid=grid,
        in_specs=[
            pl.BlockSpec((n, f_in), lambda i: (0, 0)),
            pl.BlockSpec((BR, n), lambda i: (i, 0)),
            pl.BlockSpec((f_in, c0), lambda i: (0, 0)),
            pl.BlockSpec((c0, c1), lambda i: (0, 0)),
        ],
        out_specs=pl.BlockSpec((BR, c1), lambda i: (i, 0)),
        out_shape=jax.ShapeDtypeStruct((n, c1), jnp.float32),
        scratch_shapes=[pltpu.VMEM((n, c0), jnp.float32)],
    )(x, a, W0, W1)

    out = pl.pallas_call(
        _pass2_kernel,
        grid=grid,
        in_specs=[
            pl.BlockSpec((BR, n), lambda i: (i, 0)),
            pl.BlockSpec((n, c1), lambda i: (0, 0)),
        ],
        out_specs=pl.BlockSpec((BR, c1), lambda i: (i, 0)),
        out_shape=jax.ShapeDtypeStruct((n, c1), jnp.float32),
    )(a, g)
    return out


# speedup vs baseline: 2.9274x; 2.9274x over previous
import jax
import jax.numpy as jnp
from jax.experimental import pallas as pl
from jax.experimental.pallas import tpu as pltpu

N = 10000
BR = 200

_P = jax.lax.Precision.DEFAULT


def _pass1_kernel(x_ref, a_ref, w0_ref, w1_ref, gt_ref, h0t_ref):
    @pl.when(pl.program_id(0) == 0)
    def _():
        h0t_ref[...] = jax.lax.dot_general(
            w0_ref[...], x_ref[...], (((0,), (1,)), ((), ())),
            preferred_element_type=jnp.float32, precision=_P,
        )

    zt = jax.lax.dot_general(
        h0t_ref[...], a_ref[...], (((1,), (1,)), ((), ())),
        preferred_element_type=jnp.float32, precision=_P,
    )
    zt = jnp.maximum(zt, 0.0)
    gt_ref[0] = jax.lax.dot_general(
        w1_ref[...], zt, (((0,), (0,)), ((), ())),
        preferred_element_type=jnp.float32, precision=_P,
    )


def _pass2_kernel(a_ref, gt_ref, out_ref):
    lt = jax.lax.dot_general(
        gt_ref[...], a_ref[...], (((1,), (1,)), ((), ())),
        preferred_element_type=jnp.float32, precision=_P,
    )
    m = jnp.max(lt, axis=0, keepdims=True)
    e = jnp.exp(lt - m)
    out_ref[0] = e / jnp.sum(e, axis=0, keepdims=True)


def kernel(x, a, W0, W1):
    n, f_in = x.shape
    c0 = W0.shape[1]
    c1 = W1.shape[1]
    nb = n // BR
    grid = (nb,)

    gt3 = pl.pallas_call(
        _pass1_kernel,
        grid=grid,
        in_specs=[
            pl.BlockSpec((n, f_in), lambda i: (0, 0)),
            pl.BlockSpec((BR, n), lambda i: (i, 0)),
            pl.BlockSpec((f_in, c0), lambda i: (0, 0)),
            pl.BlockSpec((c0, c1), lambda i: (0, 0)),
        ],
        out_specs=pl.BlockSpec((1, c1, BR), lambda i: (i, 0, 0)),
        out_shape=jax.ShapeDtypeStruct((nb, c1, BR), jnp.float32),
        scratch_shapes=[pltpu.VMEM((c0, n), jnp.float32)],
    )(x, a, W0, W1)
    gt = gt3.transpose(1, 0, 2).reshape(c1, n)

    out3 = pl.pallas_call(
        _pass2_kernel,
        grid=grid,
        in_specs=[
            pl.BlockSpec((BR, n), lambda i: (i, 0)),
            pl.BlockSpec((c1, n), lambda i: (0, 0)),
        ],
        out_specs=pl.BlockSpec((1, c1, BR), lambda i: (i, 0, 0)),
        out_shape=jax.ShapeDtypeStruct((nb, c1, BR), jnp.float32),
    )(a, gt)
    return out3.transpose(0, 2, 1).reshape(n, c1)


# speedup vs baseline: 2.9734x; 1.0157x over previous
import jax
import jax.numpy as jnp
from jax.experimental import pallas as pl
from jax.experimental.pallas import tpu as pltpu

N = 10000
BR = 400

_P = jax.lax.Precision.DEFAULT


def _pass1_kernel(x_ref, a_ref, w0_ref, w1_ref, gt_ref, h0t_ref):
    @pl.when(pl.program_id(0) == 0)
    def _():
        h0t_ref[...] = jax.lax.dot_general(
            w0_ref[...], x_ref[...], (((0,), (1,)), ((), ())),
            preferred_element_type=jnp.float32, precision=_P,
        )

    zt = jax.lax.dot_general(
        h0t_ref[...], a_ref[...], (((1,), (1,)), ((), ())),
        preferred_element_type=jnp.float32, precision=_P,
    )
    zt = jnp.maximum(zt, 0.0)
    gt_ref[0] = jax.lax.dot_general(
        w1_ref[...], zt, (((0,), (0,)), ((), ())),
        preferred_element_type=jnp.float32, precision=_P,
    )


def _pass2_kernel(a_ref, gt_ref, out_ref):
    lt = jax.lax.dot_general(
        gt_ref[...], a_ref[...], (((1,), (1,)), ((), ())),
        preferred_element_type=jnp.float32, precision=_P,
    )
    m = jnp.max(lt, axis=0, keepdims=True)
    e = jnp.exp(lt - m)
    out_ref[0] = e / jnp.sum(e, axis=0, keepdims=True)


def kernel(x, a, W0, W1):
    n, f_in = x.shape
    c0 = W0.shape[1]
    c1 = W1.shape[1]
    nb = n // BR
    grid = (nb,)

    gt3 = pl.pallas_call(
        _pass1_kernel,
        grid=grid,
        in_specs=[
            pl.BlockSpec((n, f_in), lambda i: (0, 0)),
            pl.BlockSpec((BR, n), lambda i: (i, 0)),
            pl.BlockSpec((f_in, c0), lambda i: (0, 0)),
            pl.BlockSpec((c0, c1), lambda i: (0, 0)),
        ],
        out_specs=pl.BlockSpec((1, c1, BR), lambda i: (i, 0, 0)),
        out_shape=jax.ShapeDtypeStruct((nb, c1, BR), jnp.float32),
        scratch_shapes=[pltpu.VMEM((c0, n), jnp.float32)],
    )(x, a, W0, W1)
    gt = gt3.transpose(1, 0, 2).reshape(c1, n)

    out3 = pl.pallas_call(
        _pass2_kernel,
        grid=grid,
        in_specs=[
            pl.BlockSpec((BR, n), lambda i: (i, 0)),
            pl.BlockSpec((c1, n), lambda i: (0, 0)),
        ],
        out_specs=pl.BlockSpec((1, c1, BR), lambda i: (i, 0, 0)),
        out_shape=jax.ShapeDtypeStruct((nb, c1, BR), jnp.float32),
    )(a, gt)
    return out3.transpose(0, 2, 1).reshape(n, c1)
